# baseline (speedup 1.0000x reference)
import functools

import jax
import jax.numpy as jnp
from jax import lax
from jax.experimental import pallas as pl
from jax.experimental.pallas import tpu as pltpu
from jax.experimental.pallas import tpu_sc as plsc

NUM_CHARS = 1000
BATCH = 4096
HIST = 50
NC = 2
NS = 16
NW = NC * NS
BPW = BATCH // NW
NBUF = 2
HP = 56
DP = 1024


@functools.partial(
    pl.kernel,
    mesh=plsc.VectorSubcoreMesh(core_axis_name="c", subcore_axis_name="s"),
    out_type=jax.ShapeDtypeStruct((BATCH, HP, DP), jnp.float32),
    scratch_types=(
        [pltpu.VMEM((BPW * HP,), jnp.int32)]
        + [pltpu.VMEM((HP, DP), jnp.float32) for _ in range(NBUF)]
        + [pltpu.SemaphoreType.DMA for _ in range(2 * NBUF)]
    ),
)
def _emb_gather(idx_hbm, table_hbm, out_hbm, idx_v, *bufs_and_sems):
    rows = bufs_and_sems[:NBUF]
    gsem = bufs_and_sems[NBUF:2 * NBUF]
    ssem = bufs_and_sems[2 * NBUF:]

    wid = lax.axis_index("s") * NC + lax.axis_index("c")
    base = wid * BPW
    pltpu.sync_copy(idx_hbm.at[pl.ds(base * HP, BPW * HP)], idx_v)

    def chunk(b, g, drain_scatter):
        dst = out_hbm.at[base + g]
        if drain_scatter:
            pltpu.make_async_copy(rows[b], dst, ssem[b]).wait()
        pltpu.async_copy(
            table_hbm.at[idx_v.at[pl.ds(g * HP, HP)]], rows[b], gsem[b]
        ).wait()
        pltpu.async_copy(rows[b], dst, ssem[b])

    for b in range(NBUF):
        chunk(b, b, drain_scatter=False)

    def group(go, carry):
        for b in range(NBUF):
            chunk(b, go * NBUF + b, drain_scatter=True)
        return carry

    lax.fori_loop(1, BPW // NBUF, group, 0)

    for b in range(NBUF):
        g = BPW - NBUF + b
        pltpu.make_async_copy(rows[b], out_hbm.at[base + g], ssem[b]).wait()


def kernel(indices, emb_weight):
    idx_pad = jnp.pad(indices.astype(jnp.int32), ((0, 0), (0, HP - HIST)))
    tbl_pad = jnp.pad(emb_weight, ((0, 0), (0, DP - NUM_CHARS)))
    out = _emb_gather(idx_pad.reshape(-1), tbl_pad)
    return out[:, :HIST, :NUM_CHARS]


# speedup vs baseline: 1.0222x; 1.0222x over previous
import functools

import jax
import jax.numpy as jnp
from jax import lax
from jax.experimental import pallas as pl
from jax.experimental.pallas import tpu as pltpu
from jax.experimental.pallas import tpu_sc as plsc

NUM_CHARS = 1000
BATCH = 4096
HIST = 50
NC = 2
NS = 16
NW = NC * NS
BPW = BATCH // NW
NBUF = 2
HP = 56
DP = 1024
NK = DP // 128


@functools.partial(
    pl.kernel,
    mesh=plsc.VectorSubcoreMesh(core_axis_name="c", subcore_axis_name="s"),
    out_type=jax.ShapeDtypeStruct((BATCH, HP, DP), jnp.float32),
    scratch_types=(
        [pltpu.VMEM((BPW * HP,), jnp.int32)]
        + [pltpu.VMEM((HP, 128), jnp.float32) for _ in range(NBUF * NK)]
        + [pltpu.SemaphoreType.DMA for _ in range(2 * NBUF)]
    ),
)
def _emb_gather(idx_hbm, *refs):
    tbls = refs[:NK]
    out_hbm = refs[NK]
    idx_v = refs[NK + 1]
    bufs = refs[NK + 2:NK + 2 + NBUF * NK]
    gsem = refs[NK + 2 + NBUF * NK:NK + 2 + NBUF * NK + NBUF]
    ssem = refs[NK + 2 + NBUF * NK + NBUF:]

    wid = lax.axis_index("s") * NC + lax.axis_index("c")
    base = wid * BPW
    pltpu.sync_copy(idx_hbm.at[pl.ds(base * HP, BPW * HP)], idx_v)

    def chunk(b, g, drain_scatter):
        dst = out_hbm.at[base + g]
        if drain_scatter:
            for k in range(NK):
                pltpu.make_async_copy(
                    bufs[b * NK + k], dst.at[:, pl.ds(k * 128, 128)], ssem[b]
                ).wait()
        idx = idx_v.at[pl.ds(g * HP, HP)]
        for k in range(NK):
            pltpu.async_copy(tbls[k].at[idx], bufs[b * NK + k], gsem[b])
        for k in range(NK):
            pltpu.make_async_copy(
                tbls[k].at[idx], bufs[b * NK + k], gsem[b]
            ).wait()
        for k in range(NK):
            pltpu.async_copy(
                bufs[b * NK + k], dst.at[:, pl.ds(k * 128, 128)], ssem[b]
            )

    for b in range(NBUF):
        chunk(b, b, drain_scatter=False)

    def group(go, carry):
        for b in range(NBUF):
            chunk(b, go * NBUF + b, drain_scatter=True)
        return carry

    lax.fori_loop(1, BPW // NBUF, group, 0)

    for b in range(NBUF):
        g = BPW - NBUF + b
        for k in range(NK):
            pltpu.make_async_copy(
                bufs[b * NK + k],
                out_hbm.at[base + g].at[:, pl.ds(k * 128, 128)],
                ssem[b],
            ).wait()


def kernel(indices, emb_weight):
    idx_pad = jnp.pad(indices.astype(jnp.int32), ((0, 0), (0, HP - HIST)))
    tbl_pad = jnp.pad(emb_weight, ((0, 0), (0, DP - NUM_CHARS)))
    octets = [tbl_pad[:, k * 128:(k + 1) * 128] for k in range(NK)]
    out = _emb_gather(idx_pad.reshape(-1), *octets)
    return out[:, :HIST, :NUM_CHARS]
